# baseline (speedup 1.0000x reference)
import functools

import jax
import jax.numpy as jnp
from jax import lax
from jax.experimental import pallas as pl
from jax.experimental.pallas import tpu as pltpu
from jax.experimental.pallas import tpu_sc as plsc

B = 16384
L = 50
ID_DIM = 32
TEXT_DIM = 32
OUT_DIM = ID_DIM + TEXT_DIM

NC, NS = 2, 16
NW = NC * NS
UPW = B // NW
GW = 128
NJ = UPW // GW
KT = 5
NB = L // KT


def _make_kernel(interpret=False):
    mesh = plsc.VectorSubcoreMesh(core_axis_name="c", subcore_axis_name="s")

    @functools.partial(
        pl.kernel,
        out_type=jax.ShapeDtypeStruct((NW, NJ, GW, OUT_DIM), jnp.float32),
        mesh=mesh,
        interpret=interpret,
        compiler_params=pltpu.CompilerParams(
            use_tc_tiling_on_sc=False, needs_layout_passes=False),
        scratch_types=[
            pltpu.VMEM((L, NJ, GW), jnp.int32),
            pltpu.VMEM((UPW // 2, L), jnp.int32),
            pltpu.VMEM((NJ, GW), jnp.int32),
            pltpu.VMEM((NJ, GW, OUT_DIM), jnp.float32),
            pltpu.VMEM((KT, GW, TEXT_DIM), jnp.float32),
            pltpu.VMEM((KT, GW, TEXT_DIM), jnp.float32),
            pltpu.VMEM((GW, TEXT_DIM), jnp.float32),
            pltpu.VMEM((UPW,), jnp.float32),
            pltpu.VMEM((UPW,), jnp.float32),
            pltpu.VMEM((TEXT_DIM,), jnp.float32),
            pltpu.SemaphoreType.DMA,
            pltpu.SemaphoreType.DMA,
            pltpu.SemaphoreType.DMA,
        ],
    )
    def user_model(idx3, tok_hbm, id_tab, txt_tab, out,
                   tok_v, tok_raw, idv, outbuf, rows_a, rows_b, acc, cnt,
                   recip, row0, sem_a, sem_b, sem2):
        w = lax.axis_index("s") * NC + lax.axis_index("c")
        base = w * UPW
        half = UPW // 2

        pltpu.sync_copy(idx3.at[w], idv)
        pltpu.sync_copy(txt_tab.at[0], row0)

        iddescs = [
            pltpu.async_copy(id_tab.at[idv.at[j]], rows_a.at[j], sem2)
            for j in range(NJ)
        ]

        iota16 = lax.iota(jnp.int32, 16)
        for r in range(2):
            pltpu.sync_copy(
                tok_hbm.at[pl.ds(base + r * half, half)], tok_raw)

            @pl.loop(0, half, unroll=2)
            def _tr(u):
                ug = r * half + u
                jb = jnp.full((16,), ug // GW, jnp.int32)
                kb = jnp.full((16,), ug % GW, jnp.int32)
                for t0 in (0, 16, 32, 34):
                    x = tok_raw[u, pl.ds(t0, 16)]
                    plsc.store_scatter(tok_v, [iota16 + t0, jb, kb], x)

        for d in iddescs:
            d.wait()
        for j in range(NJ):
            @pl.loop(0, GW, unroll=4)
            def _idcopy(u):
                outbuf[j, u, pl.ds(0, 16)] = rows_a[j, u, pl.ds(0, 16)]
                outbuf[j, u, pl.ds(16, 16)] = rows_a[j, u, pl.ds(16, 16)]

        @pl.loop(0, NJ)
        def _cnt_loop(j):
            for g in range(GW // 16):
                def body(t, c):
                    tok = tok_v[t, j, pl.ds(g * 16, 16)]
                    return c + jnp.where(tok == 0, 1.0, 0.0)
                c = lax.fori_loop(0, L, body, jnp.zeros((16,), jnp.float32),
                                  unroll=5)
                off = pl.multiple_of(j * GW + g * 16, 16)
                cnt[pl.ds(off, 16)] = c
                recip[pl.ds(off, 16)] = 1.0 / jnp.maximum(
                    jnp.float32(L) - c, 1.0)

        r0a = row0[pl.ds(0, 16)]
        r0b = row0[pl.ds(16, 16)]

        @pl.loop(0, NJ)
        def _j_loop(j):
            @pl.loop(0, GW, unroll=8)
            def _zero(u):
                z = jnp.zeros((16,), jnp.float32)
                acc[u, pl.ds(0, 16)] = z
                acc[u, pl.ds(16, 16)] = z

            def _fire(b, buf, sem):
                return [
                    pltpu.async_copy(
                        txt_tab.at[tok_v.at[b * KT + t, j]], buf.at[t], sem)
                    for t in range(KT)
                ]

            def _drain(buf, sem):
                for t in range(KT):
                    pltpu.make_async_copy(
                        txt_tab.at[tok_v.at[t, j]], buf.at[t], sem).wait()

            def _reduce_batch(buf):
                @pl.loop(0, GW, unroll=4)
                def _reduce(u):
                    h0 = acc[u, pl.ds(0, 16)]
                    h1 = acc[u, pl.ds(16, 16)]
                    for t in range(KT):
                        h0 = h0 + buf[t, u, pl.ds(0, 16)]
                        h1 = h1 + buf[t, u, pl.ds(16, 16)]
                    acc[u, pl.ds(0, 16)] = h0
                    acc[u, pl.ds(16, 16)] = h1

            _fire(0, rows_a, sem_a)

            @pl.loop(0, NB // 2)
            def _pair(p):
                _fire(2 * p + 1, rows_b, sem_b)
                _drain(rows_a, sem_a)
                _reduce_batch(rows_a)

                @pl.when(p < NB // 2 - 1)
                def _():
                    _fire(2 * p + 2, rows_a, sem_a)

                _drain(rows_b, sem_b)
                _reduce_batch(rows_b)

            @pl.loop(0, GW // 16)
            def _fin(g):
                off = pl.multiple_of(j * GW + g * 16, 16)
                cg = cnt[pl.ds(off, 16)]
                rg = recip[pl.ds(off, 16)]
                for u16 in range(16):
                    u = g * 16 + u16
                    cb = jnp.full((16,), cg[u16], jnp.float32)
                    rb = jnp.full((16,), rg[u16], jnp.float32)
                    outbuf[j, u, pl.ds(ID_DIM, 16)] = (
                        acc[u, pl.ds(0, 16)] - cb * r0a) * rb
                    outbuf[j, u, pl.ds(ID_DIM + 16, 16)] = (
                        acc[u, pl.ds(16, 16)] - cb * r0b) * rb

        pltpu.sync_copy(outbuf, out.at[w])

    return user_model


_user_model = _make_kernel()


def kernel(id_indices, token_ids, id_table, text_table):
    idx3 = id_indices.reshape(NW, NJ, GW).astype(jnp.int32)
    out = _user_model(idx3, token_ids.astype(jnp.int32), id_table, text_table)
    return out.reshape(B, OUT_DIM)


# speedup vs baseline: 1.3749x; 1.3749x over previous
import functools

import jax
import jax.numpy as jnp
from jax import lax
from jax.experimental import pallas as pl
from jax.experimental.pallas import tpu as pltpu
from jax.experimental.pallas import tpu_sc as plsc

B = 16384
L = 50
ID_DIM = 32
TEXT_DIM = 32
OUT_DIM = ID_DIM + TEXT_DIM

NC, NS = 2, 16
NW = NC * NS
UPW = B // NW
GW = 128
NJ = UPW // GW
KT = 5
NB = L // KT

_MESH = plsc.VectorSubcoreMesh(core_axis_name="c", subcore_axis_name="s")
_PARAMS = pltpu.CompilerParams(
    use_tc_tiling_on_sc=False, needs_layout_passes=False)


@functools.partial(
    pl.kernel,
    out_type=jax.ShapeDtypeStruct((NW, NJ, GW, TEXT_DIM), jnp.float32),
    mesh=_MESH,
    compiler_params=_PARAMS,
    scratch_types=[
        pltpu.VMEM((L, NJ, GW), jnp.int32),
        pltpu.VMEM((NJ, GW, TEXT_DIM), jnp.float32),
        pltpu.VMEM((KT, GW, TEXT_DIM), jnp.float32),
        pltpu.VMEM((KT, GW, TEXT_DIM), jnp.float32),
        pltpu.VMEM((GW, TEXT_DIM), jnp.float32),
        pltpu.VMEM((UPW,), jnp.float32),
        pltpu.VMEM((UPW,), jnp.float32),
        pltpu.VMEM((TEXT_DIM,), jnp.float32),
        pltpu.SemaphoreType.DMA,
        pltpu.SemaphoreType.DMA,
    ],
)
def _token_kernel(tok_t4, txt_tab, out,
                  tok_v, pooled, rows_a, rows_b, acc, cnt, recip, row0,
                  sem_a, sem_b):
    w = lax.axis_index("s") * NC + lax.axis_index("c")

    pltpu.sync_copy(tok_t4.at[:, w], tok_v)
    pltpu.sync_copy(txt_tab.at[0], row0)

    @pl.loop(0, NJ)
    def _cnt_loop(j):
        for g in range(GW // 16):
            def body(t, c):
                tok = tok_v[t, j, pl.ds(g * 16, 16)]
                return c + jnp.where(tok == 0, 1.0, 0.0)
            c = lax.fori_loop(0, L, body, jnp.zeros((16,), jnp.float32),
                              unroll=5)
            off = pl.multiple_of(j * GW + g * 16, 16)
            cnt[pl.ds(off, 16)] = c
            recip[pl.ds(off, 16)] = 1.0 / jnp.maximum(
                jnp.float32(L) - c, 1.0)

    r0a = row0[pl.ds(0, 16)]
    r0b = row0[pl.ds(16, 16)]

    @pl.loop(0, NJ)
    def _j_loop(j):
        @pl.loop(0, GW, unroll=8)
        def _zero(u):
            z = jnp.zeros((16,), jnp.float32)
            acc[u, pl.ds(0, 16)] = z
            acc[u, pl.ds(16, 16)] = z

        def _fire(b, buf, sem):
            return [
                pltpu.async_copy(
                    txt_tab.at[tok_v.at[b * KT + t, j]], buf.at[t], sem)
                for t in range(KT)
            ]

        def _drain(buf, sem):
            for t in range(KT):
                pltpu.make_async_copy(
                    txt_tab.at[tok_v.at[t, j]], buf.at[t], sem).wait()

        def _reduce_batch(buf):
            @pl.loop(0, GW, unroll=4)
            def _reduce(u):
                h0 = acc[u, pl.ds(0, 16)]
                h1 = acc[u, pl.ds(16, 16)]
                for t in range(KT):
                    h0 = h0 + buf[t, u, pl.ds(0, 16)]
                    h1 = h1 + buf[t, u, pl.ds(16, 16)]
                acc[u, pl.ds(0, 16)] = h0
                acc[u, pl.ds(16, 16)] = h1

        _fire(0, rows_a, sem_a)

        @pl.loop(0, NB // 2)
        def _pair(p):
            _fire(2 * p + 1, rows_b, sem_b)
            _drain(rows_a, sem_a)
            _reduce_batch(rows_a)

            @pl.when(p < NB // 2 - 1)
            def _():
                _fire(2 * p + 2, rows_a, sem_a)

            _drain(rows_b, sem_b)
            _reduce_batch(rows_b)

        @pl.loop(0, GW // 16)
        def _fin(g):
            off = pl.multiple_of(j * GW + g * 16, 16)
            cg = cnt[pl.ds(off, 16)]
            rg = recip[pl.ds(off, 16)]
            for u16 in range(16):
                u = g * 16 + u16
                cb = jnp.full((16,), cg[u16], jnp.float32)
                rb = jnp.full((16,), rg[u16], jnp.float32)
                pooled[j, u, pl.ds(0, 16)] = (
                    acc[u, pl.ds(0, 16)] - cb * r0a) * rb
                pooled[j, u, pl.ds(16, 16)] = (
                    acc[u, pl.ds(16, 16)] - cb * r0b) * rb

    pltpu.sync_copy(pooled, out.at[w])


@functools.partial(
    pl.kernel,
    out_type=jax.ShapeDtypeStruct((NW, NJ, GW, ID_DIM), jnp.float32),
    mesh=_MESH,
    compiler_params=_PARAMS,
    scratch_types=[
        pltpu.VMEM((NJ, GW), jnp.int32),
        pltpu.VMEM((NJ, GW, ID_DIM), jnp.float32),
        pltpu.SemaphoreType.DMA,
    ],
)
def _id_kernel(idx3, id_tab, out, idv, idrows, sem):
    w = lax.axis_index("s") * NC + lax.axis_index("c")
    pltpu.sync_copy(idx3.at[w], idv)
    descs = [
        pltpu.async_copy(id_tab.at[idv.at[j]], idrows.at[j], sem)
        for j in range(NJ)
    ]
    for d in descs:
        d.wait()
    pltpu.sync_copy(idrows, out.at[w])


def kernel(id_indices, token_ids, id_table, text_table):
    idx3 = id_indices.reshape(NW, NJ, GW).astype(jnp.int32)
    tok_t4 = token_ids.astype(jnp.int32).T.reshape(L, NW, NJ, GW)
    pooled = _token_kernel(tok_t4, text_table)
    idrows = _id_kernel(idx3, id_table)
    return jnp.concatenate(
        [idrows.reshape(B, ID_DIM), pooled.reshape(B, TEXT_DIM)], axis=1)


# speedup vs baseline: 1.4029x; 1.0203x over previous
import functools

import jax
import jax.numpy as jnp
from jax import lax
from jax.experimental import pallas as pl
from jax.experimental.pallas import tpu as pltpu
from jax.experimental.pallas import tpu_sc as plsc

B = 16384
L = 50
ID_DIM = 32
TEXT_DIM = 32
OUT_DIM = ID_DIM + TEXT_DIM

NC, NS = 2, 16
NW = NC * NS
UPW = B // NW
GW = 128
NJ = UPW // GW
KT = 5
NB = L // KT

_MESH = plsc.VectorSubcoreMesh(core_axis_name="c", subcore_axis_name="s")
_PARAMS = pltpu.CompilerParams(
    use_tc_tiling_on_sc=False, needs_layout_passes=False)


@functools.partial(
    pl.kernel,
    out_type=jax.ShapeDtypeStruct((NW, NJ, GW, TEXT_DIM), jnp.float32),
    mesh=_MESH,
    compiler_params=_PARAMS,
    scratch_types=[
        pltpu.VMEM((L, NJ, GW), jnp.int32),
        pltpu.VMEM((NJ, GW, TEXT_DIM), jnp.float32),
        pltpu.VMEM((KT, GW, TEXT_DIM), jnp.float32),
        pltpu.VMEM((KT, GW, TEXT_DIM), jnp.float32),
        pltpu.VMEM((KT, GW, TEXT_DIM), jnp.float32),
        pltpu.VMEM((KT, GW, TEXT_DIM), jnp.float32),
        pltpu.VMEM((GW, TEXT_DIM), jnp.float32),
        pltpu.VMEM((UPW,), jnp.float32),
        pltpu.VMEM((UPW,), jnp.float32),
        pltpu.VMEM((TEXT_DIM,), jnp.float32),
        pltpu.SemaphoreType.DMA,
        pltpu.SemaphoreType.DMA,
        pltpu.SemaphoreType.DMA,
        pltpu.SemaphoreType.DMA,
    ],
)
def _token_kernel(tok_t4, txt_tab, out,
                  tok_v, pooled, rows_a, rows_b, rows_c, rows_d, acc, cnt,
                  recip, row0, sem_a, sem_b, sem_c, sem_d):
    w = lax.axis_index("s") * NC + lax.axis_index("c")

    pltpu.sync_copy(tok_t4.at[:, w], tok_v)
    pltpu.sync_copy(txt_tab.at[0], row0)

    r0a = row0[pl.ds(0, 16)]
    r0b = row0[pl.ds(16, 16)]
    bufs = [(rows_a, sem_a), (rows_b, sem_b), (rows_c, sem_c), (rows_d, sem_d)]
    AHEAD = 3

    @pl.loop(0, NJ)
    def _j_loop(j):
        def _fire(b, buf, sem):
            for t in range(KT):
                pltpu.async_copy(
                    txt_tab.at[tok_v.at[b * KT + t, j]], buf.at[t], sem)

        def _drain(buf, sem):
            for t in range(KT):
                pltpu.make_async_copy(
                    txt_tab.at[tok_v.at[t, j]], buf.at[t], sem).wait()

        def _reduce_batch(buf, first):
            @pl.loop(0, GW, unroll=4)
            def _reduce(u):
                if first:
                    h0 = buf[0, u, pl.ds(0, 16)]
                    h1 = buf[0, u, pl.ds(16, 16)]
                    ts = range(1, KT)
                else:
                    h0 = acc[u, pl.ds(0, 16)]
                    h1 = acc[u, pl.ds(16, 16)]
                    ts = range(KT)
                for t in ts:
                    h0 = h0 + buf[t, u, pl.ds(0, 16)]
                    h1 = h1 + buf[t, u, pl.ds(16, 16)]
                acc[u, pl.ds(0, 16)] = h0
                acc[u, pl.ds(16, 16)] = h1

        for b in range(AHEAD):
            _fire(b, *bufs[b])

        for g in range(GW // 16):
            def body(t, c):
                tok = tok_v[t, j, pl.ds(g * 16, 16)]
                return c + jnp.where(tok == 0, 1.0, 0.0)
            c = lax.fori_loop(0, L, body, jnp.zeros((16,), jnp.float32),
                              unroll=5)
            off = pl.multiple_of(j * GW + g * 16, 16)
            cnt[pl.ds(off, 16)] = c
            recip[pl.ds(off, 16)] = 1.0 / jnp.maximum(
                jnp.float32(L) - c, 1.0)

        for b in range(NB):
            _drain(*bufs[b % 4])
            _reduce_batch(bufs[b % 4][0], first=(b == 0))
            if b + AHEAD < NB:
                _fire(b + AHEAD, *bufs[(b + AHEAD) % 4])

        @pl.loop(0, GW // 16)
        def _fin(g):
            off = pl.multiple_of(j * GW + g * 16, 16)
            cg = cnt[pl.ds(off, 16)]
            rg = recip[pl.ds(off, 16)]
            for u16 in range(16):
                u = g * 16 + u16
                cb = jnp.full((16,), cg[u16], jnp.float32)
                rb = jnp.full((16,), rg[u16], jnp.float32)
                pooled[j, u, pl.ds(0, 16)] = (
                    acc[u, pl.ds(0, 16)] - cb * r0a) * rb
                pooled[j, u, pl.ds(16, 16)] = (
                    acc[u, pl.ds(16, 16)] - cb * r0b) * rb

    pltpu.sync_copy(pooled, out.at[w])


@functools.partial(
    pl.kernel,
    out_type=jax.ShapeDtypeStruct((NW, NJ, GW, ID_DIM), jnp.float32),
    mesh=_MESH,
    compiler_params=_PARAMS,
    scratch_types=[
        pltpu.VMEM((NJ, GW), jnp.int32),
        pltpu.VMEM((NJ, GW, ID_DIM), jnp.float32),
        pltpu.SemaphoreType.DMA,
    ],
)
def _id_kernel(idx3, id_tab, out, idv, idrows, sem):
    w = lax.axis_index("s") * NC + lax.axis_index("c")
    pltpu.sync_copy(idx3.at[w], idv)
    descs = [
        pltpu.async_copy(id_tab.at[idv.at[j]], idrows.at[j], sem)
        for j in range(NJ)
    ]
    for d in descs:
        d.wait()
    pltpu.sync_copy(idrows, out.at[w])


def kernel(id_indices, token_ids, id_table, text_table):
    idx3 = id_indices.reshape(NW, NJ, GW).astype(jnp.int32)
    tok_t4 = token_ids.astype(jnp.int32).T.reshape(L, NW, NJ, GW)
    pooled = _token_kernel(tok_t4, text_table)
    idrows = _id_kernel(idx3, id_table)
    return jnp.concatenate(
        [idrows.reshape(B, ID_DIM), pooled.reshape(B, TEXT_DIM)], axis=1)
